# Initial kernel scaffold; baseline (speedup 1.0000x reference)
import jax
import jax.numpy as jnp
from jax.experimental import pallas as pl


def kernel(x, edge_index, edge_weight, edge_attr, lin1_w, mlp_w1, mlp_b1, mlp_w2, mlp_b2, lin2_w, lin2_b, lin_w, lin_b):
    raise NotImplementedError("write your pallas kernel here")


# speedup vs baseline: 1.4622x; 1.4622x over previous
import functools
import math

import jax
import jax.numpy as jnp
from jax import lax
from jax.experimental import pallas as pl
from jax.experimental.pallas import tpu as pltpu
from jax.experimental.pallas import tpu_sc as plsc

CUTOFF = 10.0
_LOG2 = math.log(2.0)


def _ssp(v):
    return jnp.maximum(v, 0.0) + jnp.log(1.0 + jnp.exp(-jnp.abs(v))) - _LOG2


def _hx_body(x_ref, w_ref, o_ref):
    o_ref[...] = jnp.dot(x_ref[...], w_ref[...], preferred_element_type=jnp.float32)


def _w_body(ea_ref, ew_ref, w1t_ref, b1_ref, w2t_ref, b2_ref, o_ref):
    h = jnp.dot(ea_ref[...], w1t_ref[...], preferred_element_type=jnp.float32)
    h = _ssp(h + b1_ref[...])
    w = jnp.dot(h, w2t_ref[...], preferred_element_type=jnp.float32) + b2_ref[...]
    c = 0.5 * (jnp.cos(ew_ref[...] * (math.pi / CUTOFF)) + 1.0)
    o_ref[...] = w * c


def _tail_body(p_ref, w2t_ref, b2_ref, wt_ref, b_ref, o_ref):
    n = o_ref.shape[0]
    npad = p_ref.shape[0] // 2
    a = p_ref[0:n, :] + p_ref[npad:npad + n, :]
    t = jnp.dot(a, w2t_ref[...], preferred_element_type=jnp.float32) + b2_ref[...]
    t = _ssp(t)
    o_ref[...] = jnp.dot(t, wt_ref[...], preferred_element_type=jnp.float32) + b_ref[...]


def _make_sc_agg(N, E, H):
    info = plsc.get_sparse_core_info()
    NC, NS, L = info.num_cores, info.num_subcores, info.num_lanes
    NW = NC * NS
    def _chunk_of(total):
        for cand in range(128, 7, -8):
            if total % cand == 0:
                return cand
        raise ValueError(total)

    assert E % NW == 0
    EPW = E // NW
    K = _chunk_of(EPW)
    NCH = EPW // K
    RPT = ((N + 128 * NS - 1) // (128 * NS)) * 128
    NP = RPT * NS
    RZ = _chunk_of(RPT)

    mesh = plsc.VectorSubcoreMesh(core_axis_name="c", subcore_axis_name="s")

    @functools.partial(
        pl.kernel,
        out_type=jax.ShapeDtypeStruct((2 * NP, H), jnp.float32),
        mesh=mesh,
        scratch_types=[
            pltpu.VMEM((K,), jnp.int32),
            pltpu.VMEM((K,), jnp.int32),
            pltpu.VMEM((K, H), jnp.float32),
            pltpu.VMEM((K, H), jnp.float32),
            pltpu.VMEM((RZ, H), jnp.float32),
            pltpu.VMEM_SHARED((NP, H), jnp.float32),
            pltpu.SemaphoreType.DMA,
        ],
    )
    def sc_agg(hx_hbm, w_hbm, src_hbm, dst_hbm, out_hbm,
               src_v, dst_v, rows_v, w_v, zb_v, acc_sh, sem):
        c = lax.axis_index("c")
        s = lax.axis_index("s")
        wid = c * NS + s

        zero = jnp.zeros((L,), jnp.float32)

        def zrow(i, _):
            for j in range(H // L):
                zb_v[i, pl.ds(j * L, L)] = zero
            return 0

        lax.fori_loop(0, RZ, zrow, 0)

        def zacc(i, _):
            pltpu.sync_copy(zb_v, acc_sh.at[pl.ds(s * RPT + i * RZ, RZ)])
            return 0

        lax.fori_loop(0, RPT // RZ, zacc, 0)
        plsc.subcore_barrier()

        base = wid * EPW

        def chunk(i, _):
            off = base + i * K
            pltpu.sync_copy(src_hbm.at[pl.ds(off, K)], src_v)
            pltpu.sync_copy(dst_hbm.at[pl.ds(off, K)], dst_v)
            cp = pltpu.async_copy(hx_hbm.at[src_v], rows_v, sem)
            pltpu.sync_copy(w_hbm.at[pl.ds(off, K)], w_v)
            cp.wait()

            def mul(e, _2):
                for j in range(H // L):
                    sl = pl.ds(j * L, L)
                    rows_v[e, sl] = rows_v[e, sl] * w_v[e, sl]
                return 0

            lax.fori_loop(0, K, mul, 0)
            pltpu.sync_copy(rows_v, acc_sh.at[dst_v], add=True)
            return 0

        lax.fori_loop(0, NCH, chunk, 0)
        plsc.subcore_barrier()

        def wout(i, _):
            r0 = s * RPT + i * RZ
            pltpu.sync_copy(acc_sh.at[pl.ds(r0, RZ)], zb_v)
            pltpu.sync_copy(zb_v, out_hbm.at[pl.ds(c * NP + r0, RZ)])
            return 0

        lax.fori_loop(0, RPT // RZ, wout, 0)

    return sc_agg


def kernel(x, edge_index, edge_weight, edge_attr, lin1_w, mlp_w1, mlp_b1,
           mlp_w2, mlp_b2, lin2_w, lin2_b, lin_w, lin_b):
    N, H = x.shape
    E, G = edge_attr.shape
    F = mlp_w1.shape[0]

    src = edge_index[0].astype(jnp.int32)
    dst = edge_index[1].astype(jnp.int32)
    ew = edge_weight.reshape(E, 1)

    hx = pl.pallas_call(
        _hx_body,
        out_shape=jax.ShapeDtypeStruct((N, H), jnp.float32),
    )(x, lin1_w.T)

    BE = 2000
    grid = E // BE
    w_full = pl.pallas_call(
        _w_body,
        grid=(grid,),
        in_specs=[
            pl.BlockSpec((BE, G), lambda i: (i, 0)),
            pl.BlockSpec((BE, 1), lambda i: (i, 0)),
            pl.BlockSpec((G, F), lambda i: (0, 0)),
            pl.BlockSpec((1, F), lambda i: (0, 0)),
            pl.BlockSpec((F, F), lambda i: (0, 0)),
            pl.BlockSpec((1, F), lambda i: (0, 0)),
        ],
        out_specs=pl.BlockSpec((BE, F), lambda i: (i, 0)),
        out_shape=jax.ShapeDtypeStruct((E, F), jnp.float32),
    )(edge_attr, ew, mlp_w1.T, mlp_b1.reshape(1, F), mlp_w2.T, mlp_b2.reshape(1, F))

    partials = _make_sc_agg(N, E, H)(hx, w_full, src, dst)

    out = pl.pallas_call(
        _tail_body,
        out_shape=jax.ShapeDtypeStruct((N, H), jnp.float32),
    )(partials, lin2_w.T, lin2_b.reshape(1, H), lin_w.T, lin_b.reshape(1, H))
    return out


# speedup vs baseline: 2.0509x; 1.4026x over previous
import functools
import math

import jax
import jax.numpy as jnp
from jax import lax
from jax.experimental import pallas as pl
from jax.experimental.pallas import tpu as pltpu
from jax.experimental.pallas import tpu_sc as plsc

CUTOFF = 10.0
_LOG2 = math.log(2.0)


def _ssp(v):
    return jnp.maximum(v, 0.0) + jnp.log(1.0 + jnp.exp(-jnp.abs(v))) - _LOG2


def _hx_body(x_ref, w_ref, o_ref):
    o_ref[...] = jnp.dot(x_ref[...], w_ref[...], preferred_element_type=jnp.float32)


def _w_body(ea_ref, ew_ref, w1t_ref, b1_ref, w2t_ref, b2_ref, o_ref, c_ref):
    h = jnp.dot(ea_ref[...], w1t_ref[...], preferred_element_type=jnp.float32)
    h = _ssp(h + b1_ref[...])
    w = jnp.dot(h, w2t_ref[...], preferred_element_type=jnp.float32) + b2_ref[...]
    o_ref[...] = w
    c_ref[...] = 0.5 * (jnp.cos(ew_ref[...] * (math.pi / CUTOFF)) + 1.0)


def _tail_body(p_ref, w2t_ref, b2_ref, wt_ref, b_ref, o_ref):
    n = o_ref.shape[0]
    npad = p_ref.shape[0] // 2
    a = p_ref[0:n, :] + p_ref[npad:npad + n, :]
    t = jnp.dot(a, w2t_ref[...], preferred_element_type=jnp.float32) + b2_ref[...]
    t = _ssp(t)
    o_ref[...] = jnp.dot(t, wt_ref[...], preferred_element_type=jnp.float32) + b_ref[...]


def _make_sc_agg(N, E, H):
    info = plsc.get_sparse_core_info()
    NC, NS, L = info.num_cores, info.num_subcores, info.num_lanes
    NW = NC * NS
    def _chunk_of(total):
        for cand in range(128, 7, -8):
            if total % cand == 0:
                return cand
        raise ValueError(total)

    assert E % NW == 0
    EPW = E // NW
    K = _chunk_of(EPW)
    NCH = EPW // K
    RPT = ((N + 128 * NS - 1) // (128 * NS)) * 128
    NP = RPT * NS
    RZ = _chunk_of(RPT)

    mesh = plsc.VectorSubcoreMesh(core_axis_name="c", subcore_axis_name="s")

    @functools.partial(
        pl.kernel,
        out_type=jax.ShapeDtypeStruct((2 * NP, H), jnp.float32),
        mesh=mesh,
        scratch_types=[
            pltpu.VMEM((K,), jnp.int32),
            pltpu.VMEM((K,), jnp.int32),
            pltpu.VMEM((K,), jnp.float32),
            pltpu.VMEM((K, H), jnp.float32),
            pltpu.VMEM((K, H), jnp.float32),
            pltpu.VMEM((RZ, H), jnp.float32),
            pltpu.VMEM_SHARED((NP, H), jnp.float32),
            pltpu.SemaphoreType.DMA,
        ],
    )
    def sc_agg(hx_hbm, w_hbm, c_hbm, src_hbm, dst_hbm, out_hbm,
               src_v, dst_v, c_v, rows_v, w_v, zb_v, acc_sh, sem):
        c = lax.axis_index("c")
        s = lax.axis_index("s")
        wid = c * NS + s

        zero = jnp.zeros((L,), jnp.float32)

        def zrow(i, _):
            for j in range(H // L):
                zb_v[i, pl.ds(j * L, L)] = zero
            return 0

        lax.fori_loop(0, RZ, zrow, 0)

        def zacc(i, _):
            pltpu.sync_copy(zb_v, acc_sh.at[pl.ds(s * RPT + i * RZ, RZ)])
            return 0

        lax.fori_loop(0, RPT // RZ, zacc, 0)
        plsc.subcore_barrier()

        base = wid * EPW

        def chunk(i, _):
            off = base + i * K
            pltpu.sync_copy(src_hbm.at[pl.ds(off, K)], src_v)
            pltpu.sync_copy(dst_hbm.at[pl.ds(off, K)], dst_v)
            cp = pltpu.async_copy(hx_hbm.at[src_v], rows_v, sem)
            pltpu.sync_copy(c_hbm.at[pl.ds(off, K)], c_v)
            pltpu.sync_copy(w_hbm.at[pl.ds(off, K)], w_v)
            cp.wait()

            def mul(q, _2):
                cvec = c_v[pl.ds(q * L, L)]
                dnums = lax.GatherDimensionNumbers(
                    offset_dims=(), collapsed_slice_dims=(0,), start_index_map=(0,))
                for t in range(L):
                    e = q * L + t
                    csplat = lax.gather(
                        cvec, jnp.full((L, 1), t, jnp.int32), dnums, (1,),
                        mode=lax.GatherScatterMode.PROMISE_IN_BOUNDS)
                    for j in range(H // L):
                        sl = pl.ds(j * L, L)
                        rows_v[e, sl] = rows_v[e, sl] * w_v[e, sl] * csplat
                return 0

            lax.fori_loop(0, K // L, mul, 0)
            pltpu.sync_copy(rows_v, acc_sh.at[dst_v], add=True)
            return 0

        lax.fori_loop(0, NCH, chunk, 0)
        plsc.subcore_barrier()

        def wout(i, _):
            r0 = s * RPT + i * RZ
            pltpu.sync_copy(acc_sh.at[pl.ds(r0, RZ)], zb_v)
            pltpu.sync_copy(zb_v, out_hbm.at[pl.ds(c * NP + r0, RZ)])
            return 0

        lax.fori_loop(0, RPT // RZ, wout, 0)

    return sc_agg


def kernel(x, edge_index, edge_weight, edge_attr, lin1_w, mlp_w1, mlp_b1,
           mlp_w2, mlp_b2, lin2_w, lin2_b, lin_w, lin_b):
    N, H = x.shape
    E, G = edge_attr.shape
    F = mlp_w1.shape[0]

    src = edge_index[0].astype(jnp.int32)
    dst = edge_index[1].astype(jnp.int32)

    hx = pl.pallas_call(
        _hx_body,
        out_shape=jax.ShapeDtypeStruct((N, H), jnp.float32),
    )(x, lin1_w.T)

    BE = 2560
    grid = E // BE
    w_full, c_full = pl.pallas_call(
        _w_body,
        grid=(grid,),
        in_specs=[
            pl.BlockSpec((BE, G), lambda i: (i, 0)),
            pl.BlockSpec((1, BE), lambda i: (0, i)),
            pl.BlockSpec((G, F), lambda i: (0, 0)),
            pl.BlockSpec((1, F), lambda i: (0, 0)),
            pl.BlockSpec((F, F), lambda i: (0, 0)),
            pl.BlockSpec((1, F), lambda i: (0, 0)),
        ],
        out_specs=[
            pl.BlockSpec((BE, F), lambda i: (i, 0)),
            pl.BlockSpec((1, BE), lambda i: (0, i)),
        ],
        out_shape=[
            jax.ShapeDtypeStruct((E, F), jnp.float32),
            jax.ShapeDtypeStruct((1, E), jnp.float32),
        ],
    )(edge_attr, edge_weight.reshape(1, E), mlp_w1.T, mlp_b1.reshape(1, F),
      mlp_w2.T, mlp_b2.reshape(1, F))

    partials = _make_sc_agg(N, E, H)(hx, w_full, c_full.reshape(E), src, dst)

    out = pl.pallas_call(
        _tail_body,
        out_shape=jax.ShapeDtypeStruct((N, H), jnp.float32),
    )(partials, lin2_w.T, lin2_b.reshape(1, H), lin_w.T, lin_b.reshape(1, H))
    return out


# speedup vs baseline: 2.2380x; 1.0912x over previous
import functools
import math

import jax
import jax.numpy as jnp
from jax import lax
from jax.experimental import pallas as pl
from jax.experimental.pallas import tpu as pltpu
from jax.experimental.pallas import tpu_sc as plsc

CUTOFF = 10.0
_LOG2 = math.log(2.0)


def _ssp(v):
    return jnp.maximum(v, 0.0) + jnp.log(1.0 + jnp.exp(-jnp.abs(v))) - _LOG2


def _hx_body(x_ref, w_ref, o_ref):
    o_ref[...] = jnp.dot(x_ref[...], w_ref[...], preferred_element_type=jnp.float32)


def _w_body(ea_ref, ew_ref, w1t_ref, b1_ref, w2t_ref, b2_ref, o_ref, c_ref):
    h = jnp.dot(ea_ref[...], w1t_ref[...], preferred_element_type=jnp.float32)
    h = _ssp(h + b1_ref[...])
    w = jnp.dot(h, w2t_ref[...], preferred_element_type=jnp.float32) + b2_ref[...]
    o_ref[...] = w
    c_ref[...] = 0.5 * (jnp.cos(ew_ref[...] * (math.pi / CUTOFF)) + 1.0)


def _tail_body(p_ref, q_ref, w2t_ref, b2_ref, wt_ref, b_ref, o_ref):
    n = o_ref.shape[0]
    npad = p_ref.shape[0] // 2
    a = (p_ref[0:n, :] + p_ref[npad:npad + n, :]
         + q_ref[0:n, :] + q_ref[npad:npad + n, :])
    t = jnp.dot(a, w2t_ref[...], preferred_element_type=jnp.float32) + b2_ref[...]
    t = _ssp(t)
    o_ref[...] = jnp.dot(t, wt_ref[...], preferred_element_type=jnp.float32) + b_ref[...]


def _make_sc_agg(N, E, H):
    info = plsc.get_sparse_core_info()
    NC, NS, L = info.num_cores, info.num_subcores, info.num_lanes
    NW = NC * NS

    def _chunk_of(total):
        for cand in range(128, 15, -16):
            if total % cand == 0:
                return cand
        raise ValueError(total)

    assert E % NW == 0
    EPW = E // NW
    K = _chunk_of(EPW)
    NCH = EPW // K
    RPT = ((N + 128 * NS - 1) // (128 * NS)) * 128
    NP = RPT * NS
    RZ = _chunk_of(RPT)

    mesh = plsc.VectorSubcoreMesh(core_axis_name="c", subcore_axis_name="s")

    @functools.partial(
        pl.kernel,
        out_type=jax.ShapeDtypeStruct((2 * NP, H), jnp.float32),
        mesh=mesh,
        scratch_types=[
            pltpu.VMEM((K,), jnp.int32),
            pltpu.VMEM((K,), jnp.int32),
            pltpu.VMEM((K,), jnp.float32),
            pltpu.VMEM((K, H), jnp.float32),
            pltpu.VMEM((K, H), jnp.float32),
            pltpu.VMEM((RZ, H), jnp.float32),
            pltpu.VMEM_SHARED((NP, H), jnp.float32),
            pltpu.SemaphoreType.DMA,
        ],
    )
    def sc_agg(hx_hbm, w_hbm, c_hbm, src_hbm, dst_hbm, out_hbm,
               src_v, dst_v, c_v, rows_v, w_v, zb_v, acc_sh, sem):
        c = lax.axis_index("c")
        s = lax.axis_index("s")
        wid = c * NS + s

        zero = jnp.zeros((L,), jnp.float32)

        def zrow(i, _):
            for j in range(H // L):
                zb_v[i, pl.ds(j * L, L)] = zero
            return 0

        lax.fori_loop(0, RZ, zrow, 0)

        def zacc(i, _):
            pltpu.sync_copy(zb_v, acc_sh.at[pl.ds(s * RPT + i * RZ, RZ)])
            return 0

        lax.fori_loop(0, RPT // RZ, zacc, 0)
        plsc.subcore_barrier()

        base = wid * EPW

        def chunk(i, _):
            off = base + i * K
            pltpu.sync_copy(src_hbm.at[pl.ds(off, K)], src_v)
            pltpu.sync_copy(dst_hbm.at[pl.ds(off, K)], dst_v)
            cp = pltpu.async_copy(hx_hbm.at[src_v], rows_v, sem)
            pltpu.sync_copy(c_hbm.at[pl.ds(off, K)], c_v)
            pltpu.sync_copy(w_hbm.at[pl.ds(off, K)], w_v)
            cp.wait()

            dnums = lax.GatherDimensionNumbers(
                offset_dims=(), collapsed_slice_dims=(0,), start_index_map=(0,))

            def mul(q, _2):
                cvec = c_v[pl.ds(q * L, L)]
                for t in range(L):
                    e = q * L + t
                    csplat = lax.gather(
                        cvec, jnp.full((L, 1), t, jnp.int32), dnums, (1,),
                        mode=lax.GatherScatterMode.PROMISE_IN_BOUNDS)
                    for j in range(H // L):
                        sl = pl.ds(j * L, L)
                        rows_v[e, sl] = rows_v[e, sl] * w_v[e, sl] * csplat
                return 0

            lax.fori_loop(0, K // L, mul, 0)
            pltpu.sync_copy(rows_v, acc_sh.at[dst_v], add=True)
            return 0

        lax.fori_loop(0, NCH, chunk, 0)
        plsc.subcore_barrier()

        def wout(i, _):
            r0 = s * RPT + i * RZ
            pltpu.sync_copy(acc_sh.at[pl.ds(r0, RZ)], zb_v)
            pltpu.sync_copy(zb_v, out_hbm.at[pl.ds(c * NP + r0, RZ)])
            return 0

        lax.fori_loop(0, RPT // RZ, wout, 0)

    return sc_agg


def kernel(x, edge_index, edge_weight, edge_attr, lin1_w, mlp_w1, mlp_b1,
           mlp_w2, mlp_b2, lin2_w, lin2_b, lin_w, lin_b):
    N, H = x.shape
    E, G = edge_attr.shape
    F = mlp_w1.shape[0]

    src = edge_index[0].astype(jnp.int32)
    dst = edge_index[1].astype(jnp.int32)

    hx = pl.pallas_call(
        _hx_body,
        out_shape=jax.ShapeDtypeStruct((N, H), jnp.float32),
    )(x, lin1_w.T)

    BE = 2560
    E1 = (E * 3 // 5 // (32 * 16 * BE)) * (32 * 16 * BE) if False else 192000
    assert E1 % BE == 0 and (E - E1) % BE == 0
    ew2 = edge_weight.reshape(1, E)
    b1r, b2r = mlp_b1.reshape(1, F), mlp_b2.reshape(1, F)
    w1t, w2t = mlp_w1.T, mlp_w2.T

    def w_seg(e0, e1):
        ob = e0 // BE
        return pl.pallas_call(
            _w_body,
            grid=((e1 - e0) // BE,),
            in_specs=[
                pl.BlockSpec((BE, G), lambda i, o=ob: (i + o, 0)),
                pl.BlockSpec((1, BE), lambda i, o=ob: (0, i + o)),
                pl.BlockSpec((G, F), lambda i: (0, 0)),
                pl.BlockSpec((1, F), lambda i: (0, 0)),
                pl.BlockSpec((F, F), lambda i: (0, 0)),
                pl.BlockSpec((1, F), lambda i: (0, 0)),
            ],
            out_specs=[
                pl.BlockSpec((BE, F), lambda i: (i, 0)),
                pl.BlockSpec((1, BE), lambda i: (0, i)),
            ],
            out_shape=[
                jax.ShapeDtypeStruct((e1 - e0, F), jnp.float32),
                jax.ShapeDtypeStruct((1, e1 - e0), jnp.float32),
            ],
        )(edge_attr, ew2, w1t, b1r, w2t, b2r)

    w_a, c_a = w_seg(0, E1)
    w_b, c_b = w_seg(E1, E)

    p_a = _make_sc_agg(N, E1, H)(hx, w_a, c_a.reshape(E1), src[:E1], dst[:E1])
    p_b = _make_sc_agg(N, E - E1, H)(hx, w_b, c_b.reshape(E - E1),
                                     src[E1:], dst[E1:])

    out = pl.pallas_call(
        _tail_body,
        out_shape=jax.ShapeDtypeStruct((N, H), jnp.float32),
    )(p_a, p_b, lin2_w.T, lin2_b.reshape(1, H), lin_w.T, lin_b.reshape(1, H))
    return out
